# baseline (speedup 1.0000x reference)
import functools

import jax
import jax.numpy as jnp
from jax import lax
from jax.experimental import pallas as pl
from jax.experimental.pallas import tpu as pltpu
from jax.experimental.pallas import tpu_sc as plsc

BATCH = 16384
BASE_DIM = 64
D1 = 32
D2 = 16
OFF1 = 100000
OFF2 = 400000

N0 = 100001
N1 = 300001
N2 = 600001

PACK = 128
FBT = 2048
N0P = 49 * FBT
N1P = 147 * FBT
N2P = 294 * FBT
G0 = FBT * BASE_DIM // PACK
G1 = FBT * D1 // PACK
G2 = FBT * D2 // PACK

NUM_CORES = 2
NUM_SUBCORES = 16
NW = NUM_CORES * NUM_SUBCORES
BPW = BATCH // NW
CHUNK = 128
NCHUNK = BPW // CHUNK
LANES = 16


def _fmt_body(n, d, xt_ref, w_ref, out_ref):
    cols = pl.program_id(0) * FBT + lax.broadcasted_iota(jnp.int32, (d, FBT), 1)
    x = jnp.where(cols < n, xt_ref[...], 0.0)
    xr = x.reshape(PACK, (FBT * d) // PACK)
    out_ref[...] = lax.dot_general(xr, w_ref[...], (((0,), (0,)), ((), ())),
                                   precision=lax.Precision.HIGHEST,
                                   preferred_element_type=jnp.float32)


def _perm(d):
    s = PACK // d
    m = jnp.arange(PACK)
    cols = d * (m % s) + m // s
    return jnp.zeros((PACK, PACK), jnp.float32).at[m, cols].set(1.0)


def _fmt(xt, d, n, np_):
    g = (FBT * d) // PACK
    return pl.pallas_call(
        functools.partial(_fmt_body, n, d),
        grid=(np_ // FBT,),
        in_specs=[
            pl.BlockSpec((d, FBT), lambda i: (0, i)),
            pl.BlockSpec((PACK, PACK), lambda i: (0, 0)),
        ],
        out_specs=pl.BlockSpec((g, PACK), lambda i: (i, 0)),
        out_shape=jax.ShapeDtypeStruct((np_ * d // PACK, PACK), jnp.float32),
    )(xt, _perm(d))


def _gather_body(ids_hbm, t0_hbm, t1_hbm, t2_hbm, e_hbm,
                 ids_v, r0i_v, r1i_v, r2i_v,
                 r0a, r1a, r2a, r0b, r1b, r2b, gsa, gsb, wsa, wsb):
    wid = lax.axis_index("s") * NUM_CORES + lax.axis_index("c")
    base = wid * BPW
    pltpu.sync_copy(ids_hbm.at[pl.ds(base, BPW)], ids_v)
    for j in range(NCHUNK):
        for t in range(CHUNK // LANES):
            off = j * CHUNK + t * LANES
            ids16 = ids_v[pl.ds(off, LANES)]
            lt1 = ids16 < OFF1
            lt2 = ids16 < OFF2
            loc0 = jnp.where(lt1, ids16 + 1, (ids16 >> 4) + 1)
            dummy1 = (ids16 >> 2) + 1
            loc1 = jnp.where(lt1, dummy1,
                             jnp.where(lt2, ids16 - (OFF1 - 1), dummy1))
            loc2 = jnp.where(lt2, (ids16 >> 1) + 1, ids16 - (OFF2 - 1))
            sl = pl.ds(t * LANES, LANES)
            r0i_v[j, sl] = ((loc0 >> 11) << 10) + (loc0 & (G0 - 1))
            r1i_v[j, sl] = ((loc1 >> 11) << 9) + (loc1 & (G1 - 1))
            r2i_v[j, sl] = ((loc2 >> 11) << 8) + (loc2 & (G2 - 1))
    bufs = [(r0a, r1a, r2a, gsa, wsa), (r0b, r1b, r2b, gsb, wsb)]
    cps = {}

    def fire(j):
        r0, r1, r2, gsem, _ = bufs[j % 2]
        cps[j] = [
            pltpu.async_copy(t0_hbm.at[r0i_v.at[j]], r0, gsem),
            pltpu.async_copy(t1_hbm.at[r1i_v.at[j]], r1, gsem),
            pltpu.async_copy(t2_hbm.at[r2i_v.at[j]], r2, gsem),
        ]

    fire(0)
    pend_w = {}
    for j in range(NCHUNK):
        for cp in cps[j]:
            cp.wait()
        if j + 1 < NCHUNK:
            if j - 1 >= 0:
                for wp in pend_w[j - 1]:
                    wp.wait()
            fire(j + 1)
        r0, r1, r2, _, wsem = bufs[j % 2]
        rows = pl.ds(base + j * CHUNK, CHUNK)
        pend_w[j] = [
            pltpu.async_copy(r0, e_hbm.at[rows, pl.ds(0, PACK)], wsem),
            pltpu.async_copy(r1, e_hbm.at[rows, pl.ds(PACK, PACK)], wsem),
            pltpu.async_copy(r2, e_hbm.at[rows, pl.ds(2 * PACK, PACK)], wsem),
        ]
    for j in (NCHUNK - 2, NCHUNK - 1):
        for wp in pend_w[j]:
            wp.wait()


def _sc_gather(ids, t0, t1, t2):
    mesh = plsc.VectorSubcoreMesh(
        core_axis_name="c", subcore_axis_name="s",
        num_cores=NUM_CORES, num_subcores=NUM_SUBCORES)
    f = pl.kernel(
        _gather_body,
        out_type=jax.ShapeDtypeStruct((BATCH, 3 * PACK), jnp.float32),
        mesh=mesh,
        compiler_params=pltpu.CompilerParams(use_tc_tiling_on_sc=False),
        scratch_types=[
            pltpu.VMEM((BPW,), jnp.int32),
            pltpu.VMEM((NCHUNK, CHUNK), jnp.int32),
            pltpu.VMEM((NCHUNK, CHUNK), jnp.int32),
            pltpu.VMEM((NCHUNK, CHUNK), jnp.int32),
            pltpu.VMEM((CHUNK, PACK), jnp.float32),
            pltpu.VMEM((CHUNK, PACK), jnp.float32),
            pltpu.VMEM((CHUNK, PACK), jnp.float32),
            pltpu.VMEM((CHUNK, PACK), jnp.float32),
            pltpu.VMEM((CHUNK, PACK), jnp.float32),
            pltpu.VMEM((CHUNK, PACK), jnp.float32),
            pltpu.SemaphoreType.DMA,
            pltpu.SemaphoreType.DMA,
            pltpu.SemaphoreType.DMA,
            pltpu.SemaphoreType.DMA,
        ],
    )
    return f(ids, t0, t1, t2)


BT = 2048


def _proj_body(ids_ref, e_ref, w1_ref, b1_ref, w2_ref, b2_ref, out_ref):
    ids = ids_ref[...]
    lt1 = ids < OFF1
    lt2 = ids < OFF2
    loc0 = jnp.where(lt1, ids + 1, (ids >> 4) + 1)
    d1v = (ids >> 2) + 1
    loc1 = jnp.where(lt1, d1v, jnp.where(lt2, ids - (OFF1 - 1), d1v))
    loc2 = jnp.where(lt2, (ids >> 1) + 1, ids - (OFF2 - 1))
    a0 = (loc0 >> 10) & 1
    a1 = (loc1 >> 9) & 3
    a2 = (loc2 >> 8) & 7
    e = e_ref[...]
    e0 = jnp.where(a0 == 0, e[:, 0:BASE_DIM], e[:, BASE_DIM:2 * BASE_DIM])
    e1 = e[:, PACK:PACK + D1]
    for a in (1, 2, 3):
        lo = PACK + D1 * a
        e1 = jnp.where(a1 == a, e[:, lo:lo + D1], e1)
    e2 = e[:, 2 * PACK:2 * PACK + D2]
    for a in range(1, 8):
        lo = 2 * PACK + D2 * a
        e2 = jnp.where(a2 == a, e[:, lo:lo + D2], e2)
    p1 = jnp.dot(e1, w1_ref[...], preferred_element_type=jnp.float32) + b1_ref[...]
    p2 = jnp.dot(e2, w2_ref[...], preferred_element_type=jnp.float32) + b2_ref[...]
    out_ref[...] = jnp.where(lt1, e0, jnp.where(lt2, p1, p2))


def _tc_project(ids2d, e, W1, b1, W2, b2):
    grid = (BATCH // BT,)
    return pl.pallas_call(
        _proj_body,
        grid=grid,
        in_specs=[
            pl.BlockSpec((BT, 1), lambda i: (i, 0)),
            pl.BlockSpec((BT, 3 * PACK), lambda i: (i, 0)),
            pl.BlockSpec((D1, BASE_DIM), lambda i: (0, 0)),
            pl.BlockSpec((1, BASE_DIM), lambda i: (0, 0)),
            pl.BlockSpec((D2, BASE_DIM), lambda i: (0, 0)),
            pl.BlockSpec((1, BASE_DIM), lambda i: (0, 0)),
        ],
        out_specs=pl.BlockSpec((BT, BASE_DIM), lambda i: (i, 0)),
        out_shape=jax.ShapeDtypeStruct((BATCH, BASE_DIM), jnp.float32),
    )(ids2d, e, W1, b1, W2, b2)


def kernel(inputs, T0, T1, W1, b1, T2, W2, b2):
    ids = inputs.astype(jnp.int32)
    t0 = _fmt(T0.T, BASE_DIM, N0, N0P)
    t1 = _fmt(T1.T, D1, N1, N1P)
    t2 = _fmt(T2.T, D2, N2, N2P)
    e = _sc_gather(ids, t0, t1, t2)
    return _tc_project(ids.reshape(BATCH, 1), e,
                       W1, b1.reshape(1, BASE_DIM), W2, b2.reshape(1, BASE_DIM))


# speedup vs baseline: 1.0647x; 1.0647x over previous
import functools

import jax
import jax.numpy as jnp
from jax import lax
from jax.experimental import pallas as pl
from jax.experimental.pallas import tpu as pltpu
from jax.experimental.pallas import tpu_sc as plsc

BATCH = 16384
BASE_DIM = 64
D1 = 32
D2 = 16
OFF1 = 100000
OFF2 = 400000

N0 = 100001
N1 = 300001
N2 = 600001

PACK = 128
FBT = 2048
N0P = 49 * FBT
N1P = 147 * FBT
N2P = 294 * FBT
G0 = FBT * BASE_DIM // PACK
G1 = FBT * D1 // PACK
G2 = FBT * D2 // PACK

NUM_CORES = 2
NUM_SUBCORES = 16
NW = NUM_CORES * NUM_SUBCORES
BPW = BATCH // NW
CHUNK = 128
NCHUNK = BPW // CHUNK
LANES = 16


def _fmt_body(n, d, xt_ref, w_ref, out_ref):
    cols = pl.program_id(0) * FBT + lax.broadcasted_iota(jnp.int32, (d, FBT), 1)
    x = jnp.where(cols < n, xt_ref[...], 0.0)
    xr = x.reshape(PACK, (FBT * d) // PACK)
    out_ref[...] = lax.dot_general(xr, w_ref[...], (((0,), (0,)), ((), ())),
                                   preferred_element_type=jnp.float32)


def _perm(d):
    s = PACK // d
    m = jnp.arange(PACK)
    cols = d * (m % s) + m // s
    return jnp.zeros((PACK, PACK), jnp.float32).at[m, cols].set(1.0)


def _fmt(xt, d, n, np_):
    g = (FBT * d) // PACK
    return pl.pallas_call(
        functools.partial(_fmt_body, n, d),
        grid=(np_ // FBT,),
        in_specs=[
            pl.BlockSpec((d, FBT), lambda i: (0, i)),
            pl.BlockSpec((PACK, PACK), lambda i: (0, 0)),
        ],
        out_specs=pl.BlockSpec((g, PACK), lambda i: (i, 0)),
        out_shape=jax.ShapeDtypeStruct((np_ * d // PACK, PACK), jnp.float32),
    )(xt, _perm(d))


def _gather_body(ids_hbm, t0_hbm, t1_hbm, t2_hbm, e_hbm,
                 ids_v, r0i_v, r1i_v, r2i_v,
                 r0a, r1a, r2a, r0b, r1b, r2b, gsa, gsb, wsa, wsb):
    wid = lax.axis_index("s") * NUM_CORES + lax.axis_index("c")
    base = wid * BPW
    pltpu.sync_copy(ids_hbm.at[pl.ds(base, BPW)], ids_v)
    for j in range(NCHUNK):
        for t in range(CHUNK // LANES):
            off = j * CHUNK + t * LANES
            ids16 = ids_v[pl.ds(off, LANES)]
            lt1 = ids16 < OFF1
            lt2 = ids16 < OFF2
            loc0 = jnp.where(lt1, ids16 + 1, (ids16 >> 4) + 1)
            dummy1 = (ids16 >> 2) + 1
            loc1 = jnp.where(lt1, dummy1,
                             jnp.where(lt2, ids16 - (OFF1 - 1), dummy1))
            loc2 = jnp.where(lt2, (ids16 >> 1) + 1, ids16 - (OFF2 - 1))
            sl = pl.ds(t * LANES, LANES)
            r0i_v[j, sl] = ((loc0 >> 11) << 10) + (loc0 & (G0 - 1))
            r1i_v[j, sl] = ((loc1 >> 11) << 9) + (loc1 & (G1 - 1))
            r2i_v[j, sl] = ((loc2 >> 11) << 8) + (loc2 & (G2 - 1))
    bufs = [(r0a, r1a, r2a, gsa, wsa), (r0b, r1b, r2b, gsb, wsb)]
    cps = {}

    def fire(j):
        r0, r1, r2, gsem, _ = bufs[j % 2]
        cps[j] = [
            pltpu.async_copy(t0_hbm.at[r0i_v.at[j]], r0, gsem),
            pltpu.async_copy(t1_hbm.at[r1i_v.at[j]], r1, gsem),
            pltpu.async_copy(t2_hbm.at[r2i_v.at[j]], r2, gsem),
        ]

    fire(0)
    pend_w = {}
    for j in range(NCHUNK):
        for cp in cps[j]:
            cp.wait()
        if j + 1 < NCHUNK:
            if j - 1 >= 0:
                for wp in pend_w[j - 1]:
                    wp.wait()
            fire(j + 1)
        r0, r1, r2, _, wsem = bufs[j % 2]
        rows = pl.ds(base + j * CHUNK, CHUNK)
        pend_w[j] = [
            pltpu.async_copy(r0, e_hbm.at[rows, pl.ds(0, PACK)], wsem),
            pltpu.async_copy(r1, e_hbm.at[rows, pl.ds(PACK, PACK)], wsem),
            pltpu.async_copy(r2, e_hbm.at[rows, pl.ds(2 * PACK, PACK)], wsem),
        ]
    for j in (NCHUNK - 2, NCHUNK - 1):
        for wp in pend_w[j]:
            wp.wait()


def _sc_gather(ids, t0, t1, t2):
    mesh = plsc.VectorSubcoreMesh(
        core_axis_name="c", subcore_axis_name="s",
        num_cores=NUM_CORES, num_subcores=NUM_SUBCORES)
    f = pl.kernel(
        _gather_body,
        out_type=jax.ShapeDtypeStruct((BATCH, 3 * PACK), jnp.float32),
        mesh=mesh,
        compiler_params=pltpu.CompilerParams(use_tc_tiling_on_sc=False),
        scratch_types=[
            pltpu.VMEM((BPW,), jnp.int32),
            pltpu.VMEM((NCHUNK, CHUNK), jnp.int32),
            pltpu.VMEM((NCHUNK, CHUNK), jnp.int32),
            pltpu.VMEM((NCHUNK, CHUNK), jnp.int32),
            pltpu.VMEM((CHUNK, PACK), jnp.float32),
            pltpu.VMEM((CHUNK, PACK), jnp.float32),
            pltpu.VMEM((CHUNK, PACK), jnp.float32),
            pltpu.VMEM((CHUNK, PACK), jnp.float32),
            pltpu.VMEM((CHUNK, PACK), jnp.float32),
            pltpu.VMEM((CHUNK, PACK), jnp.float32),
            pltpu.SemaphoreType.DMA,
            pltpu.SemaphoreType.DMA,
            pltpu.SemaphoreType.DMA,
            pltpu.SemaphoreType.DMA,
        ],
    )
    return f(ids, t0, t1, t2)


BT = 2048


def _proj_body(ids_ref, e_ref, w1_ref, b1_ref, w2_ref, b2_ref, out_ref):
    ids = ids_ref[...]
    lt1 = ids < OFF1
    lt2 = ids < OFF2
    loc0 = jnp.where(lt1, ids + 1, (ids >> 4) + 1)
    d1v = (ids >> 2) + 1
    loc1 = jnp.where(lt1, d1v, jnp.where(lt2, ids - (OFF1 - 1), d1v))
    loc2 = jnp.where(lt2, (ids >> 1) + 1, ids - (OFF2 - 1))
    a0 = (loc0 >> 10) & 1
    a1 = (loc1 >> 9) & 3
    a2 = (loc2 >> 8) & 7
    e = e_ref[...]
    e0 = jnp.where(a0 == 0, e[:, 0:BASE_DIM], e[:, BASE_DIM:2 * BASE_DIM])
    e1 = e[:, PACK:PACK + D1]
    for a in (1, 2, 3):
        lo = PACK + D1 * a
        e1 = jnp.where(a1 == a, e[:, lo:lo + D1], e1)
    e2 = e[:, 2 * PACK:2 * PACK + D2]
    for a in range(1, 8):
        lo = 2 * PACK + D2 * a
        e2 = jnp.where(a2 == a, e[:, lo:lo + D2], e2)
    p1 = jnp.dot(e1, w1_ref[...], preferred_element_type=jnp.float32) + b1_ref[...]
    p2 = jnp.dot(e2, w2_ref[...], preferred_element_type=jnp.float32) + b2_ref[...]
    out_ref[...] = jnp.where(lt1, e0, jnp.where(lt2, p1, p2))


def _tc_project(ids2d, e, W1, b1, W2, b2):
    grid = (BATCH // BT,)
    return pl.pallas_call(
        _proj_body,
        grid=grid,
        in_specs=[
            pl.BlockSpec((BT, 1), lambda i: (i, 0)),
            pl.BlockSpec((BT, 3 * PACK), lambda i: (i, 0)),
            pl.BlockSpec((D1, BASE_DIM), lambda i: (0, 0)),
            pl.BlockSpec((1, BASE_DIM), lambda i: (0, 0)),
            pl.BlockSpec((D2, BASE_DIM), lambda i: (0, 0)),
            pl.BlockSpec((1, BASE_DIM), lambda i: (0, 0)),
        ],
        out_specs=pl.BlockSpec((BT, BASE_DIM), lambda i: (i, 0)),
        out_shape=jax.ShapeDtypeStruct((BATCH, BASE_DIM), jnp.float32),
    )(ids2d, e, W1, b1, W2, b2)


def kernel(inputs, T0, T1, W1, b1, T2, W2, b2):
    ids = inputs.astype(jnp.int32)
    t0 = _fmt(T0.T, BASE_DIM, N0, N0P)
    t1 = _fmt(T1.T, D1, N1, N1P)
    t2 = _fmt(T2.T, D2, N2, N2P)
    e = _sc_gather(ids, t0, t1, t2)
    return _tc_project(ids.reshape(BATCH, 1), e,
                       W1, b1.reshape(1, BASE_DIM), W2, b2.reshape(1, BASE_DIM))


# speedup vs baseline: 1.9410x; 1.8231x over previous
import functools

import jax
import jax.numpy as jnp
from jax import lax
from jax.experimental import pallas as pl
from jax.experimental.pallas import tpu as pltpu
from jax.experimental.pallas import tpu_sc as plsc

BATCH = 16384
BASE_DIM = 64
D1 = 32
D2 = 16
OFF1 = 100000
OFF2 = 400000

N0 = 100001
N1 = 300001
N2 = 600001

PACK = 128
FBT = 8192
N0P = 13 * FBT
N1P = 37 * FBT
N2P = 74 * FBT
G0 = FBT * BASE_DIM // PACK
G1 = FBT * D1 // PACK
G2 = FBT * D2 // PACK

NUM_CORES = 2
NUM_SUBCORES = 16
NW = NUM_CORES * NUM_SUBCORES
BPW = BATCH // NW
CHUNK = 128
NCHUNK = BPW // CHUNK
LANES = 16


def _fmt_body(n, d, xt_ref, w_ref, out_ref):
    cols = pl.program_id(0) * FBT + lax.broadcasted_iota(jnp.int32, (d, FBT), 1)
    x = jnp.where(cols < n, xt_ref[...], 0.0)
    xr = x.reshape(PACK, (FBT * d) // PACK)
    out_ref[...] = lax.dot_general(xr, w_ref[...], (((0,), (0,)), ((), ())),
                                   preferred_element_type=jnp.float32)


def _perm(d):
    s = PACK // d
    m = jnp.arange(PACK)
    cols = d * (m % s) + m // s
    return jnp.zeros((PACK, PACK), jnp.float32).at[m, cols].set(1.0)


def _fmt(xt, d, n, np_):
    g = (FBT * d) // PACK
    return pl.pallas_call(
        functools.partial(_fmt_body, n, d),
        grid=(np_ // FBT,),
        in_specs=[
            pl.BlockSpec((d, FBT), lambda i: (0, i)),
            pl.BlockSpec((PACK, PACK), lambda i: (0, 0)),
        ],
        out_specs=pl.BlockSpec((g, PACK), lambda i: (i, 0)),
        out_shape=jax.ShapeDtypeStruct((np_ * d // PACK, PACK), jnp.float32),
    )(xt, _perm(d))


def _gather_body(ids_hbm, t0_hbm, t1_hbm, t2_hbm, e0_hbm, e1_hbm, e2_hbm,
                 ids_v, r0i_v, r1i_v, r2i_v,
                 r0a, r1a, r2a, r0b, r1b, r2b, gsa, gsb, wsa, wsb):
    wid = lax.axis_index("s") * NUM_CORES + lax.axis_index("c")
    base = wid * BPW
    pltpu.sync_copy(ids_hbm.at[pl.ds(base, BPW)], ids_v)
    for j in range(NCHUNK):
        for t in range(CHUNK // LANES):
            off = j * CHUNK + t * LANES
            ids16 = ids_v[pl.ds(off, LANES)]
            lt1 = ids16 < OFF1
            lt2 = ids16 < OFF2
            loc0 = jnp.where(lt1, ids16 + 1, (ids16 >> 4) + 1)
            dummy1 = (ids16 >> 2) + 1
            loc1 = jnp.where(lt1, dummy1,
                             jnp.where(lt2, ids16 - (OFF1 - 1), dummy1))
            loc2 = jnp.where(lt2, (ids16 >> 1) + 1, ids16 - (OFF2 - 1))
            sl = pl.ds(t * LANES, LANES)
            r0i_v[j, sl] = ((loc0 >> 13) << 12) + (loc0 & (G0 - 1))
            r1i_v[j, sl] = ((loc1 >> 13) << 11) + (loc1 & (G1 - 1))
            r2i_v[j, sl] = ((loc2 >> 13) << 10) + (loc2 & (G2 - 1))
    bufs = [(r0a, r1a, r2a, gsa, wsa), (r0b, r1b, r2b, gsb, wsb)]
    cps = {}

    def fire(j):
        r0, r1, r2, gsem, _ = bufs[j % 2]
        cps[j] = [
            pltpu.async_copy(t0_hbm.at[r0i_v.at[j]], r0, gsem),
            pltpu.async_copy(t1_hbm.at[r1i_v.at[j]], r1, gsem),
            pltpu.async_copy(t2_hbm.at[r2i_v.at[j]], r2, gsem),
        ]

    fire(0)
    pend_w = {}
    for j in range(NCHUNK):
        for cp in cps[j]:
            cp.wait()
        if j + 1 < NCHUNK:
            if j - 1 >= 0:
                for wp in pend_w[j - 1]:
                    wp.wait()
            fire(j + 1)
        r0, r1, r2, _, wsem = bufs[j % 2]
        rows = pl.ds(base + j * CHUNK, CHUNK)
        pend_w[j] = [
            pltpu.async_copy(r0, e0_hbm.at[rows], wsem),
            pltpu.async_copy(r1, e1_hbm.at[rows], wsem),
            pltpu.async_copy(r2, e2_hbm.at[rows], wsem),
        ]
    for j in (NCHUNK - 2, NCHUNK - 1):
        for wp in pend_w[j]:
            wp.wait()


def _sc_gather(ids, t0, t1, t2):
    mesh = plsc.VectorSubcoreMesh(
        core_axis_name="c", subcore_axis_name="s",
        num_cores=NUM_CORES, num_subcores=NUM_SUBCORES)
    f = pl.kernel(
        _gather_body,
        out_type=(
            jax.ShapeDtypeStruct((BATCH, PACK), jnp.float32),
            jax.ShapeDtypeStruct((BATCH, PACK), jnp.float32),
            jax.ShapeDtypeStruct((BATCH, PACK), jnp.float32),
        ),
        mesh=mesh,
        compiler_params=pltpu.CompilerParams(use_tc_tiling_on_sc=False),
        scratch_types=[
            pltpu.VMEM((BPW,), jnp.int32),
            pltpu.VMEM((NCHUNK, CHUNK), jnp.int32),
            pltpu.VMEM((NCHUNK, CHUNK), jnp.int32),
            pltpu.VMEM((NCHUNK, CHUNK), jnp.int32),
            pltpu.VMEM((CHUNK, PACK), jnp.float32),
            pltpu.VMEM((CHUNK, PACK), jnp.float32),
            pltpu.VMEM((CHUNK, PACK), jnp.float32),
            pltpu.VMEM((CHUNK, PACK), jnp.float32),
            pltpu.VMEM((CHUNK, PACK), jnp.float32),
            pltpu.VMEM((CHUNK, PACK), jnp.float32),
            pltpu.SemaphoreType.DMA,
            pltpu.SemaphoreType.DMA,
            pltpu.SemaphoreType.DMA,
            pltpu.SemaphoreType.DMA,
        ],
    )
    return f(ids, t0, t1, t2)


BT = 2048


def _proj_body(ids_ref, ea_ref, eb_ref, ec_ref, w1_ref, b1_ref, w2_ref, b2_ref,
               out_ref):
    ids = ids_ref[...]
    lt1 = ids < OFF1
    lt2 = ids < OFF2
    loc0 = jnp.where(lt1, ids + 1, (ids >> 4) + 1)
    d1v = (ids >> 2) + 1
    loc1 = jnp.where(lt1, d1v, jnp.where(lt2, ids - (OFF1 - 1), d1v))
    loc2 = jnp.where(lt2, (ids >> 1) + 1, ids - (OFF2 - 1))
    a0 = (loc0 >> 12) & 1
    a1 = (loc1 >> 11) & 3
    a2 = (loc2 >> 10) & 7
    ea = ea_ref[...]
    eb = eb_ref[...]
    ec = ec_ref[...]
    e0 = jnp.where(a0 == 0, ea[:, 0:BASE_DIM], ea[:, BASE_DIM:2 * BASE_DIM])
    e1 = eb[:, 0:D1]
    for a in (1, 2, 3):
        e1 = jnp.where(a1 == a, eb[:, D1 * a:D1 * a + D1], e1)
    e2 = ec[:, 0:D2]
    for a in range(1, 8):
        e2 = jnp.where(a2 == a, ec[:, D2 * a:D2 * a + D2], e2)
    p1 = jnp.dot(e1, w1_ref[...], preferred_element_type=jnp.float32) + b1_ref[...]
    p2 = jnp.dot(e2, w2_ref[...], preferred_element_type=jnp.float32) + b2_ref[...]
    out_ref[...] = jnp.where(lt1, e0, jnp.where(lt2, p1, p2))


def _tc_project(ids2d, ea, eb, ec, W1, b1, W2, b2):
    grid = (BATCH // BT,)
    return pl.pallas_call(
        _proj_body,
        grid=grid,
        in_specs=[
            pl.BlockSpec((BT, 1), lambda i: (i, 0)),
            pl.BlockSpec((BT, PACK), lambda i: (i, 0)),
            pl.BlockSpec((BT, PACK), lambda i: (i, 0)),
            pl.BlockSpec((BT, PACK), lambda i: (i, 0)),
            pl.BlockSpec((D1, BASE_DIM), lambda i: (0, 0)),
            pl.BlockSpec((1, BASE_DIM), lambda i: (0, 0)),
            pl.BlockSpec((D2, BASE_DIM), lambda i: (0, 0)),
            pl.BlockSpec((1, BASE_DIM), lambda i: (0, 0)),
        ],
        out_specs=pl.BlockSpec((BT, BASE_DIM), lambda i: (i, 0)),
        out_shape=jax.ShapeDtypeStruct((BATCH, BASE_DIM), jnp.float32),
    )(ids2d, ea, eb, ec, W1, b1, W2, b2)


def kernel(inputs, T0, T1, W1, b1, T2, W2, b2):
    ids = inputs.astype(jnp.int32)
    t0 = _fmt(T0.T, BASE_DIM, N0, N0P)
    t1 = _fmt(T1.T, D1, N1, N1P)
    t2 = _fmt(T2.T, D2, N2, N2P)
    ea, eb, ec = _sc_gather(ids, t0, t1, t2)
    return _tc_project(ids.reshape(BATCH, 1), ea, eb, ec,
                       W1, b1.reshape(1, BASE_DIM), W2, b2.reshape(1, BASE_DIM))
